# baseline (speedup 1.0000x reference)
import functools

import jax
import jax.numpy as jnp
from jax import lax
from jax.experimental import pallas as pl
from jax.experimental.pallas import tpu as pltpu
from jax.experimental.pallas import tpu_sc as plsc

DIM = 768
HIDDEN = 3072
NUM_EXPERTS = 8
TOPK = 2
T = 2048
BT = 256
NPAD = T * TOPK + NUM_EXPERTS * BT
NT = NPAD // BT
NTA = 32

NSUB = 32
TPW = T // NSUB


def _router_body(x_ref, wr_ref, br_ref, inv1_ref, inv2_ref, te_ref, used_ref):
    logits = jnp.dot(x_ref[...], wr_ref[...],
                     preferred_element_type=jnp.float32) + br_ref[...]
    eidx = lax.broadcasted_iota(jnp.int32, (T, NUM_EXPERTS), 1)

    cols = []
    for e in range(NUM_EXPERTS):
        le = logits[:, e:e + 1]
        beats = (logits > le) | ((logits == le) & (eidx < e))
        cols.append(jnp.sum(beats.astype(jnp.int32), axis=1, keepdims=True))
    rank = jnp.concatenate(cols, axis=1)

    oh_a = (rank == 0).astype(jnp.float32)
    oh_b = (rank == 1).astype(jnp.float32)

    CB = 256
    r_i = lax.broadcasted_iota(jnp.int32, (CB, CB), 0)
    c_i = lax.broadcasted_iota(jnp.int32, (CB, CB), 1)
    ltri = (c_i < r_i).astype(jnp.bfloat16)
    oh = jnp.concatenate([oh_a, oh_b], axis=1)
    parts = []
    carry = jnp.zeros((1, 2 * NUM_EXPERTS), jnp.float32)
    for b in range(T // CB):
        blk = oh[b * CB:(b + 1) * CB]
        local = jnp.dot(ltri, blk.astype(jnp.bfloat16),
                        preferred_element_type=jnp.float32)
        parts.append(local + carry)
        carry = carry + jnp.sum(blk, axis=0, keepdims=True)
    csum = jnp.concatenate(parts, axis=0)
    ca, cb = csum[:, :NUM_EXPERTS], csum[:, NUM_EXPERTS:]

    cnt_a = jnp.sum(oh_a, axis=0, keepdims=True)
    cnt_b = jnp.sum(oh_b, axis=0, keepdims=True)
    counts = cnt_a + cnt_b
    padded = jnp.ceil(counts * (1.0 / BT)) * BT
    m8r = lax.broadcasted_iota(jnp.int32, (NUM_EXPERTS, NUM_EXPERTS), 0)
    m8c = lax.broadcasted_iota(jnp.int32, (NUM_EXPERTS, NUM_EXPERTS), 1)
    sl8 = (m8r < m8c).astype(jnp.float32)
    start = jnp.dot(padded, sl8, preferred_element_type=jnp.float32)

    pos_a = jnp.sum(oh_a * (start + ca), axis=1)
    pos_b = jnp.sum(oh_b * (start + cnt_a + cb), axis=1)
    inv1_ref[...] = pos_a.astype(jnp.int32)
    inv2_ref[...] = pos_b.astype(jnp.int32)

    tb = lax.broadcasted_iota(jnp.int32, (NTA, 1), 0).astype(jnp.float32) * BT
    ge = (tb >= start).astype(jnp.int32)
    te = jnp.sum(ge, axis=1) - 1
    in_real = (tb >= start) & (tb < start + counts)
    used = jnp.sum(in_real.astype(jnp.int32), axis=1)
    te_ref[...] = jnp.maximum(te, 0)
    used_ref[...] = used


def _run_router(flat_x, Wr, br):
    return pl.pallas_call(
        _router_body,
        grid=(1,),
        in_specs=[
            pl.BlockSpec((T, DIM), lambda i: (0, 0)),
            pl.BlockSpec((DIM, NUM_EXPERTS), lambda i: (0, 0)),
            pl.BlockSpec((NUM_EXPERTS,), lambda i: (0,)),
        ],
        out_specs=[
            pl.BlockSpec((T,), lambda i: (0,)),
            pl.BlockSpec((T,), lambda i: (0,)),
            pl.BlockSpec((NTA,), lambda i: (0,)),
            pl.BlockSpec((NTA,), lambda i: (0,)),
        ],
        out_shape=[
            jax.ShapeDtypeStruct((T,), jnp.int32),
            jax.ShapeDtypeStruct((T,), jnp.int32),
            jax.ShapeDtypeStruct((NTA,), jnp.int32),
            jax.ShapeDtypeStruct((NTA,), jnp.int32),
        ],
    )(flat_x, Wr, br)


@functools.lru_cache(maxsize=None)
def _make_dispatch():
    mesh = plsc.VectorSubcoreMesh(core_axis_name="c", subcore_axis_name="s")

    @functools.partial(
        pl.kernel, mesh=mesh,
        out_type=jax.ShapeDtypeStruct((NPAD, DIM // 2), jnp.int32),
        scratch_types=[
            pltpu.VMEM((TPW, DIM // 2), jnp.int32),
            pltpu.VMEM((TPW,), jnp.int32),
            pltpu.VMEM((TPW,), jnp.int32),
            pltpu.SemaphoreType.DMA,
        ],
    )
    def dispatch(x_hbm, inv1_hbm, inv2_hbm, xs_hbm, xrows, i1_v, i2_v, sem):
        wid = lax.axis_index("s") * 2 + lax.axis_index("c")
        base = wid * TPW
        pltpu.sync_copy(x_hbm.at[pl.ds(base, TPW), :], xrows)
        pltpu.sync_copy(inv1_hbm.at[pl.ds(base, TPW)], i1_v)
        pltpu.sync_copy(inv2_hbm.at[pl.ds(base, TPW)], i2_v)
        pltpu.async_copy(xrows, xs_hbm.at[i1_v], sem).wait()
        pltpu.async_copy(xrows, xs_hbm.at[i2_v], sem).wait()

    return dispatch


def _ffn_body(te_ref, used_ref, xs_ref, w1_ref, b1_ref, w2_ref, b2_ref,
              ys_ref):
    t = pl.program_id(0)

    @pl.when(used_ref[t] > 0)
    def _():
        h = jnp.maximum(
            jnp.dot(xs_ref[...], w1_ref[0],
                    preferred_element_type=jnp.float32) + b1_ref[0, 0][None, :],
            0.0)
        y = jnp.dot(h, w2_ref[0], preferred_element_type=jnp.float32)
        ys_ref[...] = ((y + b2_ref[0, 0][None, :]) * 0.5).astype(jnp.bfloat16)


def _run_ffn(xs, W1, b1, W2, b2, te, used):
    grid_spec = pltpu.PrefetchScalarGridSpec(
        num_scalar_prefetch=2,
        grid=(NT,),
        in_specs=[
            pl.BlockSpec((BT, DIM), lambda t, te, us: (t, 0)),
            pl.BlockSpec((1, DIM, HIDDEN), lambda t, te, us: (te[t], 0, 0)),
            pl.BlockSpec((1, 1, HIDDEN), lambda t, te, us: (te[t], 0, 0)),
            pl.BlockSpec((1, HIDDEN, DIM), lambda t, te, us: (te[t], 0, 0)),
            pl.BlockSpec((1, 1, DIM), lambda t, te, us: (te[t], 0, 0)),
        ],
        out_specs=pl.BlockSpec((BT, DIM), lambda t, te, us: (t, 0)),
    )
    return pl.pallas_call(
        _ffn_body,
        grid_spec=grid_spec,
        out_shape=jax.ShapeDtypeStruct((NPAD, DIM), jnp.bfloat16),
    )(te, used, xs,
      W1, b1.reshape(NUM_EXPERTS, 1, HIDDEN),
      W2, b2.reshape(NUM_EXPERTS, 1, DIM))


@functools.lru_cache(maxsize=None)
def _make_combine():
    mesh = plsc.VectorSubcoreMesh(core_axis_name="c", subcore_axis_name="s")

    @functools.partial(
        pl.kernel, mesh=mesh,
        out_type=[jax.ShapeDtypeStruct((T, DIM // 2), jnp.int32),
                  jax.ShapeDtypeStruct((T, DIM // 2), jnp.int32)],
        scratch_types=[
            pltpu.VMEM((TPW, DIM // 2), jnp.int32),
            pltpu.VMEM((TPW, DIM // 2), jnp.int32),
            pltpu.VMEM((TPW,), jnp.int32),
            pltpu.VMEM((TPW,), jnp.int32),
            pltpu.SemaphoreType.DMA,
            pltpu.SemaphoreType.DMA,
        ],
    )
    def combine(ys_hbm, inv1_hbm, inv2_hbm, outa_hbm, outb_hbm,
                buf_a, buf_b, i1_v, i2_v, sem_a, sem_b):
        wid = lax.axis_index("s") * 2 + lax.axis_index("c")
        base = wid * TPW
        pltpu.sync_copy(inv1_hbm.at[pl.ds(base, TPW)], i1_v)
        pltpu.sync_copy(inv2_hbm.at[pl.ds(base, TPW)], i2_v)
        cp_a = pltpu.async_copy(ys_hbm.at[i1_v], buf_a, sem_a)
        cp_b = pltpu.async_copy(ys_hbm.at[i2_v], buf_b, sem_b)
        cp_a.wait()
        cp_b.wait()
        pltpu.sync_copy(buf_a, outa_hbm.at[pl.ds(base, TPW), :])
        pltpu.sync_copy(buf_b, outb_hbm.at[pl.ds(base, TPW), :])

    return combine


def _add_body(a_ref, b_ref, out_ref):
    out_ref[...] = (a_ref[...] + b_ref[...]).astype(jnp.float32)


def _run_add(ya, yb):
    return pl.pallas_call(
        _add_body,
        grid=(1,),
        in_specs=[pl.BlockSpec((T, DIM), lambda i: (0, 0)),
                  pl.BlockSpec((T, DIM), lambda i: (0, 0))],
        out_specs=pl.BlockSpec((T, DIM), lambda i: (0, 0)),
        out_shape=jax.ShapeDtypeStruct((T, DIM), jnp.float32),
    )(ya, yb)


def _bf16_to_i32(a):
    return lax.bitcast_convert_type(
        a.reshape(*a.shape[:-1], a.shape[-1] // 2, 2), jnp.int32)


def _i32_to_bf16(a):
    return lax.bitcast_convert_type(a, jnp.bfloat16).reshape(
        *a.shape[:-1], a.shape[-1] * 2)


@jax.jit
def kernel(x, Wr, br, W1, b1, W2, b2):
    B, S, D = x.shape
    flat_x = x.reshape(T, D)
    inv1, inv2, te, used = _run_router(flat_x, Wr, br)
    xb = _bf16_to_i32(flat_x.astype(jnp.bfloat16))
    xs = _i32_to_bf16(_make_dispatch()(xb, inv1, inv2))
    ys = _run_ffn(xs, W1, b1, W2, b2, te, used)
    ya, yb = _make_combine()(_bf16_to_i32(ys), inv1, inv2)
    out = _run_add(_i32_to_bf16(ya), _i32_to_bf16(yb))
    return out.reshape(B, S, D)


# speedup vs baseline: 3.0805x; 3.0805x over previous
import functools

import jax
import jax.numpy as jnp
from jax import lax
from jax.experimental import pallas as pl
from jax.experimental.pallas import tpu as pltpu
from jax.experimental.pallas import tpu_sc as plsc

DIM = 768
HIDDEN = 3072
NUM_EXPERTS = 8
TOPK = 2
T = 2048
BT = 256
NPAD = T * TOPK + NUM_EXPERTS * BT
NT = NPAD // BT
NTA = 32

NSUB = 32
TPW = T // NSUB


def _router_body(x_ref, wr_ref, br_ref, inv1_ref, inv2_ref, te_ref, used_ref):
    logits = jnp.dot(x_ref[...], wr_ref[...],
                     preferred_element_type=jnp.float32) + br_ref[...]
    eidx = lax.broadcasted_iota(jnp.int32, (T, NUM_EXPERTS), 1)

    cols = []
    for e in range(NUM_EXPERTS):
        le = logits[:, e:e + 1]
        beats = (logits > le) | ((logits == le) & (eidx < e))
        cols.append(jnp.sum(beats.astype(jnp.int32), axis=1, keepdims=True))
    rank = jnp.concatenate(cols, axis=1)

    oh_a = (rank == 0).astype(jnp.float32)
    oh_b = (rank == 1).astype(jnp.float32)

    CB = 256
    r_i = lax.broadcasted_iota(jnp.int32, (CB, CB), 0)
    c_i = lax.broadcasted_iota(jnp.int32, (CB, CB), 1)
    ltri = (c_i < r_i).astype(jnp.bfloat16)
    oh = jnp.concatenate([oh_a, oh_b], axis=1)
    parts = []
    carry = jnp.zeros((1, 2 * NUM_EXPERTS), jnp.float32)
    for b in range(T // CB):
        blk = oh[b * CB:(b + 1) * CB]
        local = jnp.dot(ltri, blk.astype(jnp.bfloat16),
                        preferred_element_type=jnp.float32)
        parts.append(local + carry)
        carry = carry + jnp.sum(blk, axis=0, keepdims=True)
    csum = jnp.concatenate(parts, axis=0)
    ca, cb = csum[:, :NUM_EXPERTS], csum[:, NUM_EXPERTS:]

    cnt_a = jnp.sum(oh_a, axis=0, keepdims=True)
    cnt_b = jnp.sum(oh_b, axis=0, keepdims=True)
    counts = cnt_a + cnt_b
    padded = jnp.ceil(counts * (1.0 / BT)) * BT
    m8r = lax.broadcasted_iota(jnp.int32, (NUM_EXPERTS, NUM_EXPERTS), 0)
    m8c = lax.broadcasted_iota(jnp.int32, (NUM_EXPERTS, NUM_EXPERTS), 1)
    sl8 = (m8r < m8c).astype(jnp.float32)
    start = jnp.dot(padded, sl8, preferred_element_type=jnp.float32)

    pos_a = jnp.sum(oh_a * (start + ca), axis=1)
    pos_b = jnp.sum(oh_b * (start + cnt_a + cb), axis=1)
    inv1_ref[...] = pos_a.astype(jnp.int32)
    inv2_ref[...] = pos_b.astype(jnp.int32)

    tb = lax.broadcasted_iota(jnp.int32, (NTA, 1), 0).astype(jnp.float32) * BT
    ge = (tb >= start).astype(jnp.int32)
    te = jnp.sum(ge, axis=1) - 1
    in_real = (tb >= start) & (tb < start + counts)
    used = jnp.sum(in_real.astype(jnp.int32), axis=1)
    te_ref[...] = jnp.maximum(te, 0)
    used_ref[...] = used


def _run_router(flat_x, Wr, br):
    return pl.pallas_call(
        _router_body,
        grid=(1,),
        in_specs=[
            pl.BlockSpec((T, DIM), lambda i: (0, 0)),
            pl.BlockSpec((DIM, NUM_EXPERTS), lambda i: (0, 0)),
            pl.BlockSpec((NUM_EXPERTS,), lambda i: (0,)),
        ],
        out_specs=[
            pl.BlockSpec((T,), lambda i: (0,)),
            pl.BlockSpec((T,), lambda i: (0,)),
            pl.BlockSpec((NTA,), lambda i: (0,)),
            pl.BlockSpec((NTA,), lambda i: (0,)),
        ],
        out_shape=[
            jax.ShapeDtypeStruct((T,), jnp.int32),
            jax.ShapeDtypeStruct((T,), jnp.int32),
            jax.ShapeDtypeStruct((NTA,), jnp.int32),
            jax.ShapeDtypeStruct((NTA,), jnp.int32),
        ],
    )(flat_x, Wr, br)


@functools.lru_cache(maxsize=None)
def _make_dispatch():
    mesh = plsc.VectorSubcoreMesh(core_axis_name="c", subcore_axis_name="s")

    @functools.partial(
        pl.kernel, mesh=mesh,
        out_type=jax.ShapeDtypeStruct((NPAD, DIM), jnp.float32),
        scratch_types=[
            pltpu.VMEM((TPW, DIM), jnp.float32),
            pltpu.VMEM((TPW,), jnp.int32),
            pltpu.VMEM((TPW,), jnp.int32),
            pltpu.SemaphoreType.DMA,
        ],
    )
    def dispatch(x_hbm, inv1_hbm, inv2_hbm, xs_hbm, xrows, i1_v, i2_v, sem):
        wid = lax.axis_index("s") * 2 + lax.axis_index("c")
        base = wid * TPW
        pltpu.sync_copy(x_hbm.at[pl.ds(base, TPW), :], xrows)
        pltpu.sync_copy(inv1_hbm.at[pl.ds(base, TPW)], i1_v)
        pltpu.sync_copy(inv2_hbm.at[pl.ds(base, TPW)], i2_v)
        pltpu.async_copy(xrows, xs_hbm.at[i1_v], sem).wait()
        pltpu.async_copy(xrows, xs_hbm.at[i2_v], sem).wait()

    return dispatch


def _ffn_body(te_ref, used_ref, xs_ref, w1_ref, b1_ref, w2_ref, b2_ref,
              ys_ref):
    t = pl.program_id(0)

    @pl.when(used_ref[t] > 0)
    def _():
        h = jnp.maximum(
            jnp.dot(xs_ref[...], w1_ref[0],
                    preferred_element_type=jnp.float32) + b1_ref[0, 0][None, :],
            0.0)
        ys_ref[...] = (jnp.dot(h, w2_ref[0], preferred_element_type=jnp.float32)
                       + b2_ref[0, 0][None, :])


def _run_ffn(xs, W1, b1, W2, b2, te, used):
    grid_spec = pltpu.PrefetchScalarGridSpec(
        num_scalar_prefetch=2,
        grid=(NT,),
        in_specs=[
            pl.BlockSpec((BT, DIM), lambda t, te, us: (t, 0)),
            pl.BlockSpec((1, DIM, HIDDEN), lambda t, te, us: (te[t], 0, 0)),
            pl.BlockSpec((1, 1, HIDDEN), lambda t, te, us: (te[t], 0, 0)),
            pl.BlockSpec((1, HIDDEN, DIM), lambda t, te, us: (te[t], 0, 0)),
            pl.BlockSpec((1, 1, DIM), lambda t, te, us: (te[t], 0, 0)),
        ],
        out_specs=pl.BlockSpec((BT, DIM), lambda t, te, us: (t, 0)),
    )
    return pl.pallas_call(
        _ffn_body,
        grid_spec=grid_spec,
        out_shape=jax.ShapeDtypeStruct((NPAD, DIM), jnp.float32),
    )(te, used, xs,
      W1, b1.reshape(NUM_EXPERTS, 1, HIDDEN),
      W2, b2.reshape(NUM_EXPERTS, 1, DIM))


@functools.lru_cache(maxsize=None)
def _make_combine():
    mesh = plsc.VectorSubcoreMesh(core_axis_name="c", subcore_axis_name="s")

    @functools.partial(
        pl.kernel, mesh=mesh,
        out_type=jax.ShapeDtypeStruct((T, DIM), jnp.float32),
        scratch_types=[
            pltpu.VMEM((TPW, DIM), jnp.float32),
            pltpu.VMEM((TPW, DIM), jnp.float32),
            pltpu.VMEM((TPW,), jnp.int32),
            pltpu.VMEM((TPW,), jnp.int32),
            pltpu.SemaphoreType.DMA,
            pltpu.SemaphoreType.DMA,
        ],
    )
    def combine(ys_hbm, inv1_hbm, inv2_hbm, out_hbm,
                buf_a, buf_b, i1_v, i2_v, sem_a, sem_b):
        wid = lax.axis_index("s") * 2 + lax.axis_index("c")
        base = wid * TPW
        pltpu.sync_copy(inv1_hbm.at[pl.ds(base, TPW)], i1_v)
        pltpu.sync_copy(inv2_hbm.at[pl.ds(base, TPW)], i2_v)
        cp_a = pltpu.async_copy(ys_hbm.at[i1_v], buf_a, sem_a)
        cp_b = pltpu.async_copy(ys_hbm.at[i2_v], buf_b, sem_b)
        cp_a.wait()
        cp_b.wait()

        def row_body(r, _):
            for j in range(DIM // 16):
                sl = pl.ds(j * 16, 16)
                buf_a[r, sl] = (buf_a[r, sl] + buf_b[r, sl]) * 0.5
            return 0

        lax.fori_loop(0, TPW, row_body, 0)
        pltpu.sync_copy(buf_a, out_hbm.at[pl.ds(base, TPW), :])

    return combine


@jax.jit
def kernel(x, Wr, br, W1, b1, W2, b2):
    B, S, D = x.shape
    flat_x = x.reshape(T, D)
    inv1, inv2, te, used = _run_router(flat_x, Wr, br)
    xs = _make_dispatch()(flat_x, inv1, inv2)
    ys = _run_ffn(xs, W1, b1, W2, b2, te, used)
    out = _make_combine()(ys, inv1, inv2)
    return out.reshape(B, S, D)
